# Initial kernel scaffold; baseline (speedup 1.0000x reference)
import jax
import jax.numpy as jnp
from jax.experimental import pallas as pl


def kernel(h, x, edge_index, edge_attr, params):
    raise NotImplementedError("write your pallas kernel here")


# speedup vs baseline: 2.6135x; 2.6135x over previous
import functools
import jax
import jax.numpy as jnp
from jax import lax
from jax.experimental import pallas as pl
from jax.experimental.pallas import tpu as pltpu
from jax.experimental.pallas import tpu_sc as plsc

NN = 10000
EE = 160000
HH = 128
NC, NS, LL = 2, 16, 16
NW = NC * NS
CH = 128
NBLKP = 1280
EP = NBLKP * CH
TPW = NBLKP // NW
NNA = NN + 16
GRP = CH // LL
CHR = 64
TPW2 = TPW * (CH // CHR)
GRP2 = CHR // LL

_mesh = plsc.VectorSubcoreMesh(core_axis_name="c", subcore_axis_name="s",
                               num_cores=NC, num_subcores=NS)
_f32 = jnp.float32
_cpar = pltpu.CompilerParams(needs_layout_passes=False)


def _ids():
    cid = lax.axis_index("c")
    sid = lax.axis_index("s")
    return cid, sid, sid * NC + cid


def _w0(wid):
    return pl.multiple_of(wid * TPW, 8)


def _sqrt16(y):
    yi = plsc.bitcast(y, jnp.int32)
    s = plsc.bitcast((yi >> 1) + 0x1FBD1DF6, _f32)
    for _ in range(3):
        s = 0.5 * (s + y / s)
    return s


RSPAN = 632
RLAST = NN - (NS - 1) * RSPAN


def _span_chunks(rows_v, acc_sh, out, cid, sid, direction):
    start = pl.multiple_of(sid * RSPAN, 8)
    bt = rows_v.shape[0]

    def move(o, c):
        if direction == 'zero':
            pltpu.sync_copy(rows_v.at[pl.ds(0, c)],
                            acc_sh.at[pl.ds(start + o, c)])
        else:
            pltpu.sync_copy(acc_sh.at[pl.ds(start + o, c)],
                            rows_v.at[pl.ds(0, c)])
            pltpu.sync_copy(rows_v.at[pl.ds(0, c)],
                            out.at[pl.ds(cid * NN + start + o, c)])

    for o in range(0, 512, bt):
        move(o, bt)

    @pl.when(sid < NS - 1)
    def _rest():
        for o in range(512, RSPAN, bt):
            move(o, min(bt, RSPAN - o))

    @pl.when(sid == NS - 1)
    def _last():
        for o in range(512, RLAST, bt):
            move(o, min(bt, RLAST - o))


def _acc_zero(rows_v, acc_sh, znh, sid):
    pltpu.sync_copy(znh.at[pl.ds(0, rows_v.shape[0])], rows_v)
    _span_chunks(rows_v, acc_sh, None, 0, sid, 'zero')


def _acc_out(rows_v, acc_sh, out, cid, sid):
    _span_chunks(rows_v, acc_sh, out, cid, sid, 'out')


def _lagged_scatter(blocks_fn, sems):
    prev = None
    for t in range(TPW):
        cur = blocks_fn(t, sems[t % 2])
        if prev is not None:
            for h in prev:
                h.wait()
        prev = cur
    for h in prev:
        h.wait()


@functools.partial(
    pl.kernel,
    out_type=(jax.ShapeDtypeStruct((NBLKP, CH), _f32),
              jax.ShapeDtypeStruct((NC * 6 * NN,), _f32),
              jax.ShapeDtypeStruct((NW * 80,), _f32)),
    mesh=_mesh,
    compiler_params=_cpar,
    scratch_types=[
        pltpu.VMEM((NNA,), _f32), pltpu.VMEM((NNA,), _f32), pltpu.VMEM((NNA,), _f32),
        pltpu.VMEM((TPW, CH), jnp.int32), pltpu.VMEM((TPW, CH), jnp.int32),
        pltpu.VMEM((TPW, CH), _f32), pltpu.VMEM((TPW, CH), _f32),
        pltpu.VMEM((CH,), _f32), pltpu.VMEM((80,), _f32),
        pltpu.VMEM_SHARED((NNA,), _f32), pltpu.VMEM_SHARED((NNA,), _f32),
        pltpu.VMEM_SHARED((NNA,), _f32), pltpu.VMEM_SHARED((NNA,), _f32),
        pltpu.VMEM_SHARED((NNA,), _f32), pltpu.VMEM_SHARED((NNA,), _f32),
        pltpu.SemaphoreType.DMA, pltpu.SemaphoreType.DMA,
    ])
def _k0(row2, col2, a2, xc0, xc1, xc2, zn, r_out, seg_out, mom_out,
        x0, x1, x2, ridx_a, cidx_a, ab_a, rb_a, onesb, mom,
        s_cc, s_rc, s_ac, s_cr, s_rr, s_ar, semA, semB):
    cid, sid, wid = _ids()
    w0 = _w0(wid)

    @pl.when(sid == 0)
    def _zero():
        pltpu.sync_copy(zn, x0.at[pl.ds(0, NN)])
        for s in (s_cc, s_rc, s_ac, s_cr, s_rr, s_ar):
            pltpu.sync_copy(x0.at[pl.ds(0, NN)], s.at[pl.ds(0, NN)])

    pltpu.sync_copy(row2.at[pl.ds(w0, TPW)], ridx_a)
    pltpu.sync_copy(col2.at[pl.ds(w0, TPW)], cidx_a)
    pltpu.sync_copy(a2.at[pl.ds(w0, TPW)], ab_a)
    pltpu.sync_copy(xc0, x0)
    pltpu.sync_copy(xc1, x1)
    pltpu.sync_copy(xc2, x2)
    for g in range(GRP):
        onesb[pl.ds(g * LL, LL)] = jnp.ones((LL,), _f32)
    for i in range(5):
        mom[pl.ds(i * LL, LL)] = jnp.zeros((LL,), _f32)
    plsc.subcore_barrier()

    def blk(t, carry):
        for g in range(GRP):
            sl = pl.ds(g * LL, LL)
            ri = ridx_a[t, sl]
            ci = cidx_a[t, sl]
            dx = plsc.load_gather(x0, [ci]) - plsc.load_gather(x0, [ri])
            dy = plsc.load_gather(x1, [ci]) - plsc.load_gather(x1, [ri])
            dz = plsc.load_gather(x2, [ci]) - plsc.load_gather(x2, [ri])
            r16 = dx * dx + dy * dy + dz * dz
            rb_a[t, sl] = r16
            a16 = ab_a[t, sl]
            for mi, mval in enumerate((r16, a16, r16 * r16, a16 * a16,
                                       r16 * a16)):
                msl = pl.ds(mi * LL, LL)
                mom[msl] = mom[msl] + mval
        return carry

    lax.fori_loop(0, TPW, blk, 0)

    def scat(t, sem):
        ci = cidx_a.at[t]
        ri = ridx_a.at[t]
        return [
            pltpu.async_copy(onesb, s_cc.at[ci], sem, add=True),
            pltpu.async_copy(rb_a.at[t], s_rc.at[ci], sem, add=True),
            pltpu.async_copy(ab_a.at[t], s_ac.at[ci], sem, add=True),
            pltpu.async_copy(onesb, s_cr.at[ri], sem, add=True),
            pltpu.async_copy(rb_a.at[t], s_rr.at[ri], sem, add=True),
            pltpu.async_copy(ab_a.at[t], s_ar.at[ri], sem, add=True),
        ]

    _lagged_scatter(scat, (semA, semB))
    pltpu.sync_copy(rb_a, r_out.at[pl.ds(w0, TPW)])
    pltpu.sync_copy(mom, mom_out.at[pl.ds(pl.multiple_of(wid * 80, 8), 80)])
    plsc.subcore_barrier()

    @pl.when(sid == 0)
    def _out():
        for j, s in enumerate((s_cc, s_rc, s_ac, s_cr, s_rr, s_ar)):
            pltpu.sync_copy(s.at[pl.ds(0, NN)], x0.at[pl.ds(0, NN)])
            pltpu.sync_copy(x0.at[pl.ds(0, NN)],
                            seg_out.at[pl.ds((cid * 6 + j) * NN, NN)])


@functools.partial(
    pl.kernel,
    out_type=jax.ShapeDtypeStruct((NC * NN, HH), _f32),
    mesh=_mesh,
    compiler_params=_cpar,
    scratch_types=[
        pltpu.VMEM((TPW, CH), jnp.int32), pltpu.VMEM((TPW, CH), jnp.int32),
        pltpu.VMEM((CH, HH), _f32), pltpu.VMEM((CH, HH), _f32),
        pltpu.VMEM_SHARED((NNA, HH), _f32),
        pltpu.SemaphoreType.DMA, pltpu.SemaphoreType.DMA,
    ])
def _k1(row2, col2, tbl, znh, gout, ridx_a, cidx_a, rows0, rows1,
        acc_sh, sem0, sem1):
    cid, sid, wid = _ids()
    w0 = _w0(wid)
    _acc_zero(rows0, acc_sh, znh, sid)
    pltpu.sync_copy(row2.at[pl.ds(w0, TPW)], ridx_a)
    pltpu.sync_copy(col2.at[pl.ds(w0, TPW)], cidx_a)
    plsc.subcore_barrier()

    bufs = (rows0, rows1)
    sems = (sem0, sem1)
    hs = [None, None]
    hs[0] = pltpu.async_copy(tbl.at[ridx_a.at[0]], bufs[0], sems[0])
    for t in range(TPW):
        cur = t % 2
        if t + 1 < TPW:
            hs[1 - cur] = pltpu.async_copy(tbl.at[ridx_a.at[t + 1]],
                                           bufs[1 - cur], sems[1 - cur])
        hs[cur].wait()
        pltpu.sync_copy(bufs[cur], acc_sh.at[cidx_a.at[t]], add=True)

    plsc.subcore_barrier()
    _acc_out(rows0, acc_sh, gout, cid, sid)


@functools.partial(
    pl.kernel,
    out_type=(jax.ShapeDtypeStruct((NC * NN, HH), _f32),
              jax.ShapeDtypeStruct((NC * 3 * NN,), _f32)),
    mesh=_mesh,
    compiler_params=_cpar,
    scratch_types=[
        pltpu.VMEM((NNA,), _f32), pltpu.VMEM((NNA,), _f32),
        pltpu.VMEM((CH,), jnp.int32), pltpu.VMEM((CH,), jnp.int32),
        pltpu.VMEM((CH,), _f32), pltpu.VMEM((CH,), _f32),
        pltpu.VMEM((CH,), _f32), pltpu.VMEM((CH,), _f32), pltpu.VMEM((CH,), _f32),
        pltpu.VMEM((16,), _f32),
        pltpu.VMEM((CH, HH), _f32),
        pltpu.VMEM_SHARED((NNA, HH), _f32),
        pltpu.VMEM_SHARED((NNA,), _f32), pltpu.VMEM_SHARED((NNA,), _f32),
        pltpu.VMEM_SHARED((NNA,), _f32),
        pltpu.SemaphoreType.DMA,
    ])
def _k2(rowi, coli, r_e, a_e, sacN, sarN, tbl, consts, znh, zn,
        wout, sout,
        sac_v, sar_v, ridx, cidx, rbuf, abuf, attb, vrb, vab, cv,
        rows_v, acc_sh, s1_sh, sr_sh, sa_sh, sem):
    cid, sid, wid = _ids()

    @pl.when(sid == 0)
    def _zero():
        pltpu.sync_copy(zn, sac_v.at[pl.ds(0, NN)])
        for s in (s1_sh, sr_sh, sa_sh):
            pltpu.sync_copy(sac_v.at[pl.ds(0, NN)], s.at[pl.ds(0, NN)])

    pltpu.sync_copy(sacN, sac_v)
    pltpu.sync_copy(sarN, sar_v)
    pltpu.sync_copy(consts, cv)
    c16 = cv[...]
    kr = c16[0]
    ka = c16[1]
    k0c = c16[2]
    _acc_zero(rows_v, acc_sh, znh, sid)
    plsc.subcore_barrier()

    def blk(t, carry):
        base = pl.multiple_of((wid * TPW + t) * CH, 8)
        pltpu.sync_copy(rowi.at[pl.ds(base, CH)], ridx)
        pltpu.sync_copy(coli.at[pl.ds(base, CH)], cidx)
        pltpu.sync_copy(r_e.at[pl.ds(base, CH)], rbuf)
        pltpu.sync_copy(a_e.at[pl.ds(base, CH)], abuf)
        gath = pltpu.async_copy(tbl.at[ridx], rows_v, sem)
        for g in range(GRP):
            sl = pl.ds(g * LL, LL)
            ri = ridx[sl]
            ci = cidx[sl]
            z = (plsc.load_gather(sac_v, [ci]) + plsc.load_gather(sar_v, [ri])
                 + rbuf[sl] * kr + abuf[sl] * ka + k0c)
            att = 1.0 / (1.0 + jnp.exp(-z))
            attb[sl] = att
            vrb[sl] = att * rbuf[sl]
            vab[sl] = att * abuf[sl]
        gath.wait()

        def rowscale(gi, c2):
            att16 = attb[pl.ds(gi * LL, LL)]
            for l in range(LL):
                at = att16[l]
                ii = gi * LL + l
                for j in range(HH // LL):
                    sl2 = pl.ds(j * LL, LL)
                    rows_v[ii, sl2] = rows_v[ii, sl2] * at
            return c2

        lax.fori_loop(0, GRP, rowscale, 0)
        pltpu.sync_copy(rows_v, acc_sh.at[cidx], add=True)
        pltpu.sync_copy(attb, s1_sh.at[cidx], add=True)
        pltpu.sync_copy(vrb, sr_sh.at[cidx], add=True)
        pltpu.sync_copy(vab, sa_sh.at[cidx], add=True)
        return carry

    lax.fori_loop(0, TPW, blk, 0)
    plsc.subcore_barrier()
    _acc_out(rows_v, acc_sh, wout, cid, sid)

    @pl.when(sid == 0)
    def _out():
        for j, s in enumerate((s1_sh, sr_sh, sa_sh)):
            pltpu.sync_copy(s.at[pl.ds(0, NN)], sac_v.at[pl.ds(0, NN)])
            pltpu.sync_copy(sac_v.at[pl.ds(0, NN)],
                            sout.at[pl.ds((cid * 3 + j) * NN, NN)])


@functools.partial(
    pl.kernel,
    out_type=jax.ShapeDtypeStruct((NC * 4 * NN,), _f32),
    mesh=_mesh,
    compiler_params=_cpar,
    scratch_types=[
        pltpu.VMEM((NNA,), _f32), pltpu.VMEM((NNA,), _f32),
        pltpu.VMEM((NNA,), _f32), pltpu.VMEM((NNA,), _f32), pltpu.VMEM((NNA,), _f32),
        pltpu.VMEM((TPW, CH), jnp.int32), pltpu.VMEM((TPW, CH), jnp.int32),
        pltpu.VMEM((TPW, CH), _f32), pltpu.VMEM((TPW, CH), _f32),
        pltpu.VMEM((TPW, CH), _f32), pltpu.VMEM((TPW, CH), _f32),
        pltpu.VMEM((TPW, CH), _f32), pltpu.VMEM((TPW, CH), _f32),
        pltpu.VMEM((16,), _f32),
        pltpu.VMEM_SHARED((NNA,), _f32), pltpu.VMEM_SHARED((NNA,), _f32),
        pltpu.VMEM_SHARED((NNA,), _f32), pltpu.VMEM_SHARED((NNA,), _f32),
        pltpu.SemaphoreType.DMA, pltpu.SemaphoreType.DMA,
    ])
def _k3(row2, col2, r2, a2, xc0, xc1, xc2, sc3N, sr3N, consts, zn, pq_out,
        sc3_v, sr3_v, x0, x1, x2, ridx_a, cidx_a, rb_a, ab_a,
        v0, v1, v2, v3, cv, p_sh, q0_sh, q1_sh, q2_sh, semA, semB):
    pq_sh = (p_sh, q0_sh, q1_sh, q2_sh)
    cid, sid, wid = _ids()
    w0 = _w0(wid)

    @pl.when(sid == 0)
    def _zero():
        pltpu.sync_copy(zn, sc3_v.at[pl.ds(0, NN)])
        for j in range(4):
            pltpu.sync_copy(sc3_v.at[pl.ds(0, NN)], pq_sh[j].at[pl.ds(0, NN)])

    pltpu.sync_copy(row2.at[pl.ds(w0, TPW)], ridx_a)
    pltpu.sync_copy(col2.at[pl.ds(w0, TPW)], cidx_a)
    pltpu.sync_copy(r2.at[pl.ds(w0, TPW)], rb_a)
    pltpu.sync_copy(a2.at[pl.ds(w0, TPW)], ab_a)
    pltpu.sync_copy(sc3N, sc3_v)
    pltpu.sync_copy(sr3N, sr3_v)
    pltpu.sync_copy(xc0, x0)
    pltpu.sync_copy(xc1, x1)
    pltpu.sync_copy(xc2, x2)
    pltpu.sync_copy(consts, cv)
    c16 = cv[...]
    rr = c16[0]
    ra = c16[1]
    r0 = c16[2]
    plsc.subcore_barrier()

    def blk(t, carry):
        for g in range(GRP):
            sl = pl.ds(g * LL, LL)
            ri = ridx_a[t, sl]
            ci = cidx_a[t, sl]
            r16 = rb_a[t, sl]
            t3 = (plsc.load_gather(sc3_v, [ci]) + plsc.load_gather(sr3_v, [ri])
                  + r16 * rr + ab_a[t, sl] * ra + r0)
            gg = 1.0 / (_sqrt16(r16 + 1e-8) + 1.0)
            gt = gg * t3
            v0[t, sl] = gt
            v1[t, sl] = gt * plsc.load_gather(x0, [ri])
            v2[t, sl] = gt * plsc.load_gather(x1, [ri])
            v3[t, sl] = gt * plsc.load_gather(x2, [ri])
        return carry

    lax.fori_loop(0, TPW, blk, 0)

    def scat(t, sem):
        ci = cidx_a.at[t]
        return [pltpu.async_copy(v.at[t], s.at[ci], sem, add=True)
                for v, s in zip((v0, v1, v2, v3), pq_sh)]

    _lagged_scatter(scat, (semA, semB))
    plsc.subcore_barrier()

    @pl.when(sid == 0)
    def _out():
        for j in range(4):
            pltpu.sync_copy(pq_sh[j].at[pl.ds(0, NN)], sc3_v.at[pl.ds(0, NN)])
            pltpu.sync_copy(sc3_v.at[pl.ds(0, NN)],
                            pq_out.at[pl.ds((cid * 4 + j) * NN, NN)])


def _affine_body(x_ref, w_ref, b_ref, o_ref):
    o_ref[...] = (jnp.dot(x_ref[...], w_ref[...],
                          preferred_element_type=_f32) + b_ref[...])


def _tc_affine(x, w, b, bm=1000):
    m, k = x.shape
    p = w.shape[1]
    return pl.pallas_call(
        _affine_body,
        grid=(m // bm,),
        in_specs=[pl.BlockSpec((bm, k), lambda i: (i, 0)),
                  pl.BlockSpec((k, p), lambda i: (0, 0)),
                  pl.BlockSpec((1, p), lambda i: (0, 0))],
        out_specs=pl.BlockSpec((bm, p), lambda i: (i, 0)),
        out_shape=jax.ShapeDtypeStruct((m, p), _f32),
    )(x, w, b.reshape(1, p))


def _xtw_body(x_ref, w_ref, o_ref):
    @pl.when(pl.program_id(0) == 0)
    def _init():
        o_ref[...] = jnp.zeros_like(o_ref)

    o_ref[...] += lax.dot_general(
        x_ref[...], w_ref[...], (((0,), (0,)), ((), ())),
        preferred_element_type=_f32)


def _tc_xtw(x, w, bk=2000):
    k, m = x.shape
    p = w.shape[1]
    return pl.pallas_call(
        _xtw_body,
        grid=(k // bk,),
        in_specs=[pl.BlockSpec((bk, m), lambda i: (i, 0)),
                  pl.BlockSpec((bk, p), lambda i: (i, 0))],
        out_specs=pl.BlockSpec((m, p), lambda i: (0, 0)),
        out_shape=jax.ShapeDtypeStruct((m, p), _f32),
    )(x, w)


def _pad16(mat):
    return jnp.concatenate([mat, jnp.zeros((NNA - NN, mat.shape[1]), _f32)], 0)


def _pad16v(v):
    return jnp.concatenate([v, jnp.zeros((NNA - NN,), _f32)])


def kernel(h, x, edge_index, edge_attr, params):
    h = h.astype(_f32)
    x = x.astype(_f32)
    pad = jnp.full((EP - EE,), NN, jnp.int32)
    row2 = jnp.concatenate([edge_index[0].astype(jnp.int32), pad]).reshape(NBLKP, CH)
    col2 = jnp.concatenate([edge_index[1].astype(jnp.int32), pad]).reshape(NBLKP, CH)
    a2 = jnp.concatenate([edge_attr[:, 0].astype(_f32),
                          jnp.zeros((EP - EE,), _f32)]).reshape(NBLKP, CH)
    xtp = jnp.concatenate([x.T, jnp.zeros((3, NNA - NN), _f32)], axis=1)
    xc0, xc1, xc2 = xtp[0], xtp[1], xtp[2]
    zn = jnp.zeros((NN,), _f32)
    znh = jnp.zeros((NN, HH), _f32)
    E = float(EE)

    r2, segp, momp = _k0(row2, col2, a2, xc0, xc1, xc2, zn)
    znh = znh + r2[NBLKP - 1, CH - 1]
    segp = segp.reshape(NC, 6, NN)
    seg = segp[0] + segp[1]
    cc, Rc, Ac, cr, Rr, Ar = seg[0], seg[1], seg[2], seg[3], seg[4], seg[5]
    momv = jnp.sum(momp.reshape(NW, 5, LL), axis=(0, 2))
    Sr, Sa, Srr, Saa, Sra = momv[0], momv[1], momv[2], momv[3], momv[4]
    onesN = jnp.ones((NN,), _f32)
    Xstat = jnp.concatenate(
        [jnp.stack([cc, cr, Rc, Rr, Ac, Ar, onesN], axis=1),
         jnp.zeros((NN, 121), _f32)], axis=1)

    for i in range(2):
        p = params['gcl%d' % i]
        W0, b0 = p['ew0'], p['eb0']
        Wc, Wr = W0[:, :HH], W0[:, HH:2 * HH]
        wra, waa = W0[:, 2 * HH], W0[:, 2 * HH + 1]
        hcr = _tc_affine(h, jnp.concatenate([Wc.T, Wr.T], axis=1),
                         jnp.zeros((2 * HH,), _f32))
        hc, hr = hcr[:, :HH], hcr[:, HH:]
        gp = _k1(row2, col2, _pad16(hr), znh)
        G = gp[:NN] + gp[NN:]
        Wt = jnp.concatenate([hc, hr, hc * hc, hr * hr, hc * G], axis=1)
        S = _tc_xtw(Xstat, Wt)
        cchc, crhr = S[0, 0:HH], S[1, HH:2 * HH]
        Rchc, Rrhr = S[2, 0:HH], S[3, HH:2 * HH]
        Achc, Arhr = S[4, 0:HH], S[5, HH:2 * HH]
        cchc2, crhr2 = S[0, 2 * HH:3 * HH], S[1, 3 * HH:4 * HH]
        sumhcG = S[6, 4 * HH:5 * HH]
        m = (cchc + crhr) / E + (Sr / E) * wra + (Sa / E) * waa + b0
        q = (cchc2 + crhr2 + 2.0 * sumhcG
             + 2.0 * (wra * Rchc + waa * Achc + b0 * cchc)
             + 2.0 * (wra * Rrhr + waa * Arhr + b0 * crhr)
             + Srr * wra ** 2 + Saa * waa ** 2 + E * b0 ** 2
             + 2.0 * (Sra * wra * waa + Sr * wra * b0 + Sa * waa * b0)) / E
        var = q - m * m
        s_bn = p['ebn_w'] / jnp.sqrt(var + 1e-5)
        t_bn = p['ebn_b'] - m * s_bn
        W1s = p['ew1'] * s_bn[None, :]
        d2 = W1s @ b0 + t_bn @ p['ew1'].T + p['eb1']
        aw, ab = p['aw'][0], p['ab'][0]
        wr2 = W1s @ wra
        wa2 = W1s @ waa
        W2 = jnp.concatenate(
            [W1s.T, (W1s.T @ aw)[:, None], jnp.zeros((HH, 127), _f32)], axis=1)
        both = _tc_affine(jnp.concatenate([hc, hr], axis=0), W2,
                          jnp.zeros((2 * HH,), _f32))
        hc2, sac = both[:NN, :HH], both[:NN, HH]
        hr2, sar = both[NN:, :HH], both[NN:, HH]
        kr = aw @ wr2
        ka = aw @ wa2
        k0c = aw @ d2 + ab
        consts = jnp.zeros((16,), _f32).at[0].set(kr).at[1].set(ka).at[2].set(k0c)
        wp, sp = _k2(row2.reshape(EP), col2.reshape(EP),
                     r2.reshape(EP), a2.reshape(EP),
                     _pad16v(sac), _pad16v(sar),
                     _pad16(hr2), consts, znh, zn)
        Wm = wp[:NN] + wp[NN:]
        sp = sp.reshape(NC, 3, NN)
        s3 = sp[0] + sp[1]
        S1, SR, SA = s3[0], s3[1], s3[2]
        agg = (hc2 * S1[:, None] + Wm + jnp.outer(SR, wr2)
               + jnp.outer(SA, wa2) + jnp.outer(S1, d2)) / 100.0
        o1 = _tc_affine(jnp.concatenate([h, agg], axis=1), p['nw0'].T, p['nb0'])
        o2 = _tc_affine(o1, p['nw1'].T, p['nb1'])
        h = h + o2

    q_ = params['equiv']
    W0, b0 = q_['cw0'], q_['cb0']
    Wc, Wr = W0[:, :HH], W0[:, HH:2 * HH]
    wra, waa = W0[:, 2 * HH], W0[:, 2 * HH + 1]
    hcrq = _tc_affine(h, jnp.concatenate([Wc.T, Wr.T], axis=1),
                      jnp.zeros((2 * HH,), _f32))
    hcq, hrq = hcrq[:, :HH], hcrq[:, HH:]
    gp = _k1(row2, col2, _pad16(hrq), znh)
    Gq = gp[:NN] + gp[NN:]
    X512 = jnp.concatenate([Xstat, hcq * cc[:, None], hrq * cr[:, None], hcq],
                           axis=1)
    W384 = jnp.concatenate([hcq, hrq, Gq], axis=1)
    S2 = _tc_xtw(X512, W384)
    cchcq, crhrq = S2[0, 0:HH], S2[1, HH:2 * HH]
    Rchcq, Rrhrq = S2[2, 0:HH], S2[3, HH:2 * HH]
    Achcq, Arhrq = S2[4, 0:HH], S2[5, HH:2 * HH]
    Mu = S2[HH:2 * HH, 0:HH]
    Mv = S2[2 * HH:3 * HH, HH:2 * HH]
    Muv = S2[3 * HH:4 * HH, 2 * HH:3 * HH]
    m1 = (cchcq + crhrq) / E + (Sr / E) * wra + (Sa / E) * waa + b0
    us = (jnp.outer(Rchcq, wra) + jnp.outer(Achcq, waa) + jnp.outer(cchcq, b0))
    vs = (jnp.outer(Rrhrq, wra) + jnp.outer(Arhrq, waa) + jnp.outer(crhrq, b0))
    ss = (Srr * jnp.outer(wra, wra) + Saa * jnp.outer(waa, waa)
          + E * jnp.outer(b0, b0)
          + Sra * (jnp.outer(wra, waa) + jnp.outer(waa, wra))
          + Sr * (jnp.outer(wra, b0) + jnp.outer(b0, wra))
          + Sa * (jnp.outer(waa, b0) + jnp.outer(b0, waa)))
    M = (Mu + Mv + Muv + Muv.T + us + us.T + vs + vs.T + ss) / E
    Cov1 = M - jnp.outer(m1, m1)
    var1 = jnp.diag(Cov1)
    s0 = q_['cbn0_w'] / jnp.sqrt(var1 + 1e-5)
    t0 = q_['cbn0_b'] - m1 * s0
    A1 = q_['cw1'] * s0[None, :]
    c1v = t0 @ q_['cw1'].T + q_['cb1']
    m2 = m1 @ A1.T + c1v
    var2 = jnp.sum((A1 @ Cov1) * A1, axis=1)
    s1 = q_['cbn1_w'] / jnp.sqrt(var2 + 1e-5)
    t1b = q_['cbn1_b'] - m2 * s1
    w2v = q_['cw2'][0]
    wq = s1 * w2v
    w3 = A1.T @ wq
    c3 = c1v @ wq + t1b @ w2v
    scr = _tc_affine(jnp.concatenate([hcq, hrq], axis=0),
                     jnp.concatenate([w3[:, None], jnp.zeros((HH, 127), _f32)],
                                     axis=1),
                     jnp.zeros((HH,), _f32))
    sc3 = scr[:NN, 0]
    sr3 = scr[NN:, 0]
    rr3 = wra @ w3
    ra3 = waa @ w3
    r03 = b0 @ w3 + c3
    consts3 = jnp.zeros((16,), _f32).at[0].set(rr3).at[1].set(ra3).at[2].set(r03)
    pq = _k3(row2, col2, r2, a2, xc0, xc1, xc2, _pad16v(sc3), _pad16v(sr3),
             consts3, zn)
    pq = pq.reshape(NC, 4, NN)
    pqs = pq[0] + pq[1]
    P, Q = pqs[0], jnp.stack([pqs[1], pqs[2], pqs[3]], axis=1)
    x = x + (x * P[:, None] - Q) / 100.0
    return (h, x)
